# Initial kernel scaffold; baseline (speedup 1.0000x reference)
import jax
import jax.numpy as jnp
from jax.experimental import pallas as pl


def kernel(x, edge_index, batch, node, topo, lin0_W, lin0_b, sage_Wl, sage_bl, sage_Wr, gru_Wih, gru_bih, gru_Whh, gru_bhh, lstm_Wih, lstm_bih, lstm_Whh, lstm_bhh, node_emb, topo_emb, fc1_W, fc1_b, fc11_W, fc11_b, fc12_W, fc12_b, fc13_W, fc13_b):
    raise NotImplementedError("write your pallas kernel here")


# speedup vs baseline: 3.8897x; 3.8897x over previous
import functools

import jax
import jax.numpy as jnp
from jax import lax
from jax.experimental import pallas as pl
from jax.experimental.pallas import tpu as pltpu
from jax.experimental.pallas import tpu_sc as plsc

N = 10000
E = 160000
DIM = 256
HALF = 128
B = 64
NODE_NUM = 1000
TOPO_NUM = 100

NC = 2
NS = 16
CH = 125
C = 80
NP = 10240
RPT = NP // NS

DEGW = 16


def _make_spmm():
    mesh = plsc.VectorSubcoreMesh(core_axis_name="c", subcore_axis_name="s")
    out_type = jax.ShapeDtypeStruct((NC, NP, HALF), jnp.float32)
    scratch = [
        pltpu.VMEM_SHARED((NP, HALF), jnp.float32),
        pltpu.VMEM((CH, C), jnp.int32),
        pltpu.VMEM((CH, C), jnp.int32),
        pltpu.VMEM((C, HALF), jnp.float32),
        pltpu.SemaphoreType.DMA,
    ]

    def body(x_hbm, src_hbm, dst_hbm, z_hbm, agg_hbm,
             acc, src_v, dst_v, rows_v, sem):
        c = lax.axis_index("c")
        s = lax.axis_index("s")
        w = c * NS + s
        r0 = s * RPT
        pltpu.sync_copy(z_hbm.at[pl.ds(r0, RPT)], acc.at[pl.ds(r0, RPT)])
        pltpu.sync_copy(src_hbm.at[w], src_v)
        pltpu.sync_copy(dst_hbm.at[w], dst_v)
        plsc.subcore_barrier()

        def chunk(j, carry):
            pltpu.async_copy(x_hbm.at[src_v.at[j]], rows_v, sem).wait()
            pltpu.sync_copy(rows_v, acc.at[dst_v.at[j]], add=True)
            return carry

        lax.fori_loop(0, CH, chunk, 0)
        plsc.subcore_barrier()
        pltpu.sync_copy(acc.at[pl.ds(r0, RPT)],
                        agg_hbm.at[c, pl.ds(r0, RPT)])

    return pl.kernel(body, out_type=out_type, mesh=mesh,
                     scratch_types=scratch)


def _make_deg():
    mesh = plsc.VectorSubcoreMesh(core_axis_name="c", subcore_axis_name="s")
    out_type = jax.ShapeDtypeStruct((NC, NP, HALF), jnp.float32)
    scratch = [
        pltpu.VMEM_SHARED((NP, HALF), jnp.float32),
        pltpu.VMEM((CH, C), jnp.int32),
        pltpu.VMEM((C, HALF), jnp.float32),
    ]

    def body(dst_hbm, z_hbm, ones_hbm, deg_hbm, accd, dst_v, ones_v):
        c = lax.axis_index("c")
        s = lax.axis_index("s")
        r0 = s * RPT
        pltpu.sync_copy(z_hbm.at[pl.ds(r0, RPT)], accd.at[pl.ds(r0, RPT)])
        pltpu.sync_copy(dst_hbm.at[s], dst_v)
        pltpu.sync_copy(ones_hbm, ones_v)
        plsc.subcore_barrier()

        def chunk(j, carry):
            pltpu.sync_copy(ones_v, accd.at[dst_v.at[j]], add=True)
            return carry

        @pl.when(c == 0)
        def _():
            lax.fori_loop(0, 63, chunk, 0)

        @pl.when(c == 1)
        def _():
            lax.fori_loop(63, CH, chunk, 0)

        plsc.subcore_barrier()
        pltpu.sync_copy(accd.at[pl.ds(r0, RPT)],
                        deg_hbm.at[c, pl.ds(r0, RPT)])

    return pl.kernel(body, out_type=out_type, mesh=mesh,
                     scratch_types=scratch)


_sc_cache = {}


def _get_sc(name):
    if name not in _sc_cache:
        _sc_cache[name] = _make_spmm() if name == "spmm" else _make_deg()
    return _sc_cache[name]


def _edge_aggregate(xflat, src2, dst2, zeros):
    return _get_sc("spmm")(xflat, src2, dst2, zeros)


def _edge_degree(dst, zeros, ones):
    return _get_sc("deg")(dst, zeros, ones)


RB = 2000
G = N // RB


def _lin0_body(x_ref, w_ref, b_ref, o_ref):
    y = jnp.maximum(
        jnp.dot(x_ref[...], w_ref[...], preferred_element_type=jnp.float32, precision=lax.Precision.HIGHEST)
        + b_ref[...], 0.0)
    o_ref[0] = y[:, :HALF]
    o_ref[1] = y[:, HALF:]


def _layer_body(h_ref, a_ref, deg_ref, wlT_ref, bl_ref, wrT_ref,
                wihT_ref, bih_ref, whhT_ref, bhh_ref, o_ref):
    inv = 1.0 / jnp.maximum(deg_ref[0][:, 0:1] + deg_ref[1][:, 0:1], 1.0)
    out = jnp.concatenate([h_ref[0], h_ref[1]], axis=1)
    agg = jnp.concatenate([a_ref[0], a_ref[1]], axis=1) * inv
    conv = (jnp.dot(agg, wlT_ref[...], preferred_element_type=jnp.float32, precision=lax.Precision.HIGHEST)
            + bl_ref[...]
            + jnp.dot(out, wrT_ref[...], preferred_element_type=jnp.float32, precision=lax.Precision.HIGHEST))
    m = jnp.maximum(conv, 0.0) + out
    gx = jnp.dot(m, wihT_ref[...], preferred_element_type=jnp.float32, precision=lax.Precision.HIGHEST) + bih_ref[...]
    gh = jnp.dot(out, whhT_ref[...], preferred_element_type=jnp.float32, precision=lax.Precision.HIGHEST) + bhh_ref[...]
    r = jax.nn.sigmoid(gx[:, :DIM] + gh[:, :DIM])
    z = jax.nn.sigmoid(gx[:, DIM:2 * DIM] + gh[:, DIM:2 * DIM])
    n = jnp.tanh(gx[:, 2 * DIM:] + r * gh[:, 2 * DIM:])
    h = (1.0 - z) * n + z * out
    o_ref[0] = h[:, :HALF]
    o_ref[1] = h[:, HALF:]


def _head_body(h_ref, batch_ref, node_ref, topo_ref, nemb_ref, temb_ref,
               wihT_ref, bih_ref, whhT_ref, bhh_ref,
               fc1T_ref, b1_ref, fc11T_ref, b11_ref,
               fc12T_ref, b12_ref, fc13T_ref, b13_ref, o_ref):
    out = jnp.concatenate([h_ref[0], h_ref[1]], axis=1)
    gid = lax.broadcasted_iota(jnp.int32, (1, B), 1)
    Mb = batch_ref[...] == gid
    Mf = Mb.astype(jnp.float32)
    q_star = jnp.zeros((B, 2 * DIM), jnp.float32)
    hl = jnp.zeros((B, DIM), jnp.float32)
    cl = jnp.zeros((B, DIM), jnp.float32)
    for _ in range(2):
        gates = (jnp.dot(q_star, wihT_ref[...], preferred_element_type=jnp.float32)
                 + bih_ref[...]
                 + jnp.dot(hl, whhT_ref[...], preferred_element_type=jnp.float32)
                 + bhh_ref[...])
        ig = jax.nn.sigmoid(gates[:, :DIM])
        fg = jax.nn.sigmoid(gates[:, DIM:2 * DIM])
        gg = jnp.tanh(gates[:, 2 * DIM:3 * DIM])
        og = jax.nn.sigmoid(gates[:, 3 * DIM:])
        cl = fg * cl + ig * gg
        hl = og * jnp.tanh(cl)
        q = hl
        esc = lax.dot_general(out, q, (((1,), (1,)), ((), ())),
                              preferred_element_type=jnp.float32)
        em = jnp.where(Mb, esc, -1e30)
        emax = jnp.max(em, axis=0, keepdims=True)
        a = jnp.exp(em - emax) * Mf
        asum = jnp.sum(a, axis=0, keepdims=True)
        an = a / jnp.clip(asum, 1e-16, None)
        rt = lax.dot_general(an, out, (((0,), (0,)), ((), ())),
                             preferred_element_type=jnp.float32)
        q_star = jnp.concatenate([q, rt], axis=1)
    nid = lax.broadcasted_iota(jnp.int32, (1, NODE_NUM), 1)
    nh = jnp.dot((node_ref[...] == nid).astype(jnp.float32), nemb_ref[...],
                 preferred_element_type=jnp.float32)
    tid = lax.broadcasted_iota(jnp.int32, (1, TOPO_NUM), 1)
    th = jnp.dot((topo_ref[...] == tid).astype(jnp.float32), temb_ref[...],
                 preferred_element_type=jnp.float32)
    g = jnp.concatenate([q_star, nh, th], axis=1)
    x1 = jnp.maximum(jnp.dot(g, fc1T_ref[...], preferred_element_type=jnp.float32)
                     + b1_ref[...], 0.0)
    x1 = jnp.maximum(jnp.dot(x1, fc11T_ref[...], preferred_element_type=jnp.float32)
                     + b11_ref[...], 0.0)
    x1 = jnp.maximum(jnp.dot(x1, fc12T_ref[...], preferred_element_type=jnp.float32)
                     + b12_ref[...], 0.0)
    o_ref[...] = jnp.dot(x1, fc13T_ref[...], preferred_element_type=jnp.float32) + b13_ref[...]


def _full(shape):
    return pl.BlockSpec(shape, lambda i: tuple(0 for _ in shape))


def _lin0(x, w0T, b0):
    return pl.pallas_call(
        _lin0_body,
        grid=(G,),
        in_specs=[
            pl.BlockSpec((RB, DIM), lambda i: (i, 0)),
            _full((DIM, DIM)),
            _full((1, DIM)),
        ],
        out_specs=pl.BlockSpec((NC, RB, HALF), lambda i: (0, i, 0)),
        out_shape=jax.ShapeDtypeStruct((NC, N, HALF), jnp.float32),
    )(x, w0T, b0)


def _layer(h, agg, deg, wlT, bl, wrT, wihT, bih, whhT, bhh):
    return pl.pallas_call(
        _layer_body,
        grid=(G,),
        in_specs=[
            pl.BlockSpec((NC, RB, HALF), lambda i: (0, i, 0)),
            pl.BlockSpec((NC, RB, HALF), lambda i: (0, i, 0)),
            pl.BlockSpec((NC, RB, HALF), lambda i: (0, i, 0)),
            _full((DIM, DIM)),
            _full((1, DIM)),
            _full((DIM, DIM)),
            _full((DIM, 3 * DIM)),
            _full((1, 3 * DIM)),
            _full((DIM, 3 * DIM)),
            _full((1, 3 * DIM)),
        ],
        out_specs=pl.BlockSpec((NC, RB, HALF), lambda i: (0, i, 0)),
        out_shape=jax.ShapeDtypeStruct((NC, N, HALF), jnp.float32),
    )(h, agg, deg, wlT, bl, wrT, wihT, bih, whhT, bhh)


def _head(h, batch2, node2, topo2, node_emb, topo_emb,
          lwihT, lbih, lwhhT, lbhh,
          fc1T, b1, fc11T, b11, fc12T, b12, fc13T, b13):
    return pl.pallas_call(
        _head_body,
        out_shape=jax.ShapeDtypeStruct((B, 1), jnp.float32),
    )(h, batch2, node2, topo2, node_emb, topo_emb,
      lwihT, lbih, lwhhT, lbhh, fc1T, b1, fc11T, b11, fc12T, b12, fc13T, b13)


def kernel(x, edge_index, batch, node, topo, lin0_W, lin0_b, sage_Wl, sage_bl,
           sage_Wr, gru_Wih, gru_bih, gru_Whh, gru_bhh, lstm_Wih, lstm_bih,
           lstm_Whh, lstm_bhh, node_emb, topo_emb, fc1_W, fc1_b, fc11_W,
           fc11_b, fc12_W, fc12_b, fc13_W, fc13_b):
    src = edge_index[0].reshape(NS, CH, C)
    dst = edge_index[1].reshape(NS, CH, C)
    src2 = jnp.concatenate([src[None], src[None] + N], axis=0).reshape(NC * NS, CH, C)
    dst2 = jnp.broadcast_to(dst[None], (NC, NS, CH, C)).reshape(NC * NS, CH, C)
    zeros = jnp.zeros((NP, HALF), jnp.float32)
    ones = jnp.ones((C, HALF), jnp.float32)

    w0T = lin0_W.T
    b0 = lin0_b.reshape(1, DIM)
    wlT = sage_Wl.T
    bl = sage_bl.reshape(1, DIM)
    wrT = sage_Wr.T
    wihT = gru_Wih.T
    bih = gru_bih.reshape(1, 3 * DIM)
    whhT = gru_Whh.T
    bhh = gru_bhh.reshape(1, 3 * DIM)
    lwihT = lstm_Wih.T
    lbih = lstm_bih.reshape(1, 4 * DIM)
    lwhhT = lstm_Whh.T
    lbhh = lstm_bhh.reshape(1, 4 * DIM)
    fc1T = fc1_W.T
    b1 = fc1_b.reshape(1, 2 * DIM)
    fc11T = fc11_W.T
    b11 = fc11_b.reshape(1, DIM)
    fc12T = fc12_W.T
    b12 = fc12_b.reshape(1, DIM)
    fc13T = fc13_W.T
    b13 = fc13_b.reshape(1, 1)
    batch2 = batch.reshape(N, 1)
    node2 = node.reshape(B, 1)
    topo2 = topo.reshape(B, 1)

    h = _lin0(x, w0T, b0)

    deg = _edge_degree(dst, zeros, ones)
    for layer_i in range(3):
        xflat = h.reshape(NC * N, HALF)
        agg = _edge_aggregate(xflat, src2, dst2, zeros)
        h = _layer(h, agg, deg, wlT, bl, wrT, wihT, bih, whhT, bhh)

    return _head(h, batch2, node2, topo2, node_emb, topo_emb,
                 lwihT, lbih, lwhhT, lbhh,
                 fc1T, b1, fc11T, b11, fc12T, b12, fc13T, b13)
